# baseline (speedup 1.0000x reference)
import jax
import jax.numpy as jnp
from jax.experimental import pallas as pl

_DIM = 64
_HEADS = 8
_DH = 64
_INNER = _HEADS * _DH
_N = 128
_CBS = 4
_NC = _N // _CBS
_WIN = 2
_FF = 256
_BATCH = 512

_BB = 2
_NEG = -1e9
_SCALE = _DH ** -0.5
_HI = jax.lax.Precision.HIGHEST


def _ln_t(xt, g_col, b_col, eps=1e-5):
    m = xt.mean(0, keepdims=True)
    v = ((xt - m) ** 2).mean(0, keepdims=True)
    return (xt - m) / jnp.sqrt(v + eps) * g_col + b_col


def _tabnsa_kernel(
    x_ref, x2_ref,
    w_emb_ref, b_emb_ref, w_embt_ref, b_embt_ref,
    wqst_ref, wk_ref, wvt_ref,
    wgt_ref, bgt_ref,
    wot_ref,
    ln1g_ref, ln1b_ref, ln2g_ref, ln2b_ref,
    wt1_ref, bt1r_ref, wt2_ref, bt2r_ref,
    wc1t_ref, bc1c_ref, wc2t_ref, bc2c_ref,
    wh1t_ref, bh1c_ref, wh2t_ref, bh2_ref,
    o_ref,
):
    f32 = jnp.float32
    kk = jax.lax.broadcasted_iota(jnp.int32, (_N, _N), 0)
    qq = jax.lax.broadcasted_iota(jnp.int32, (_N, _N), 1)
    causal = kk <= qq
    winm = causal & ((qq - kk) < _WIN)
    cmask = ((kk // _CBS) * _CBS + (_CBS - 1)) <= qq
    negfill = jnp.where(qq[0:1, :] < (_CBS - 1), 0.0, _NEG)
    kk32 = jax.lax.broadcasted_iota(jnp.int32, (_NC, _N), 0)
    qq32 = jax.lax.broadcasted_iota(jnp.int32, (_NC, _N), 1)
    cmask_blk = (kk32 * _CBS + (_CBS - 1)) <= qq32
    poolm = jnp.where(qq32 // _CBS == kk32, 0.25, 0.0).astype(f32)
    poolt = jnp.where((kk // _CBS)[:, :_NC] == qq[:, :_NC], 0.25,
                      0.0).astype(f32)
    expm = jnp.where((kk // _CBS)[:, :_NC] == qq[:, :_NC], 1.0,
                     0.0).astype(f32)
    neg_big = jnp.finfo(f32).min

    xcol = x_ref[...].reshape(_BB * _N, 1)
    e = xcol * w_emb_ref[...] + b_emb_ref[...]
    et = jnp.concatenate(
        [w_embt_ref[...] * x2_ref[b] + b_embt_ref[...] for b in range(_BB)],
        axis=1)

    k_all = jnp.dot(e, wk_ref[...])
    qt_all = jnp.dot(wqst_ref[...], et)
    vt_all = jnp.dot(wvt_ref[...], et)
    gt_all = jax.nn.sigmoid(jnp.dot(wgt_ref[...], et) + bgt_ref[...])

    outs = []
    for b in range(_BB):
        r0 = b * _N
        kb = k_all[r0:r0 + _N]
        qtb = qt_all[:, r0:r0 + _N]
        vtb = vt_all[:, r0:r0 + _N]
        gtb = gt_all[:, r0:r0 + _N]
        kcb = jnp.dot(poolm, kb, precision=_HI)
        vctb = jnp.dot(vtb, poolt)

        attn_heads = []
        for h in range(_HEADS):
            s0 = h * _DH
            qt = qtb[s0:s0 + _DH]
            vt = vtb[s0:s0 + _DH]
            vct = vctb[s0:s0 + _DH]
            s_full = jnp.dot(kb[:, s0:s0 + _DH], qt)
            sc_blk = jnp.dot(kcb[:, s0:s0 + _DH], qt)

            sc_m = jnp.where(cmask_blk, sc_blk, negfill)
            ec = jnp.exp(sc_m)
            dc = jnp.sum(ec, axis=0, keepdims=True)

            m1 = jnp.max(sc_m, axis=0, keepdims=True)
            sc_ne = jnp.where(sc_m == m1, neg_big, sc_m)
            m2 = jnp.max(sc_ne, axis=0, keepdims=True)
            fsel_blk = jnp.where((sc_m == m1) | (sc_m == m2), 1.0, 0.0)
            fsel = jnp.dot(expm, fsel_blk)

            e_c = jnp.exp(jnp.where(causal, s_full, _NEG))

            w_f = e_c * fsel
            w_w = jnp.where(winm, e_c, 0.0)
            d_f = jnp.sum(w_f, axis=0, keepdims=True)
            d_w = jnp.sum(w_w, axis=0, keepdims=True)

            g0 = gtb[h:h + 1]
            g1 = gtb[_HEADS + h:_HEADS + h + 1]
            g2 = gtb[2 * _HEADS + h:2 * _HEADS + h + 1]
            p_fw = (g1 / d_f) * w_f + (g2 / d_w) * w_w
            p_c = (g0 / dc) * ec
            attn_heads.append(jnp.dot(vt, p_fw) + jnp.dot(vct, p_c))
        outs.append(jnp.concatenate(attn_heads, axis=0))

    attn_t = jnp.concatenate(outs, axis=1)
    attn_out = jnp.dot(wot_ref[...], attn_t)

    t = _ln_t(et, ln1g_ref[...], ln1b_ref[...])
    tmix = []
    for b in range(_BB):
        tb = t[:, b * _N:(b + 1) * _N]
        a1 = jax.nn.gelu(jnp.dot(tb, wt1_ref[...]) + bt1r_ref[...])
        tmix.append(jnp.dot(a1, wt2_ref[...]) + bt2r_ref[...])
    y = et + jnp.concatenate(tmix, axis=1)
    c_in = _ln_t(y, ln2g_ref[...], ln2b_ref[...])
    c1 = jax.nn.gelu(jnp.dot(wc1t_ref[...], c_in) + bc1c_ref[...])
    cmix = jnp.dot(wc2t_ref[...], c1) + bc2c_ref[...]
    mix = y + cmix

    s_all = attn_out + mix
    pooled = jnp.concatenate(
        [jnp.mean(s_all[:, b * _N:(b + 1) * _N], axis=1, keepdims=True)
         for b in range(_BB)], axis=1)
    h1 = jax.nn.gelu(jnp.dot(wh1t_ref[...], pooled) + bh1c_ref[...])
    out_t = jnp.dot(wh2t_ref[...], h1) + bh2_ref[...]
    out = jnp.concatenate([out_t[:, b:b + 1] for b in range(_BB)], axis=0)
    o_ref[...] = out.reshape(_BB, 1, 1)


@jax.jit
def kernel(x, params):
    p = params
    xr = x.reshape(_BATCH, _N, 1)
    xr2 = x.reshape(_BATCH, 1, _N)
    row2 = lambda a: a.reshape(1, -1)
    col2 = lambda a: a.reshape(-1, 1)
    ins = (
        xr, xr2,
        p['W_emb'], row2(p['b_emb']), col2(p['W_emb']), col2(p['b_emb']),
        (p['Wq'] * _SCALE).T, p['Wk'], p['Wv'].T,
        p['Wg'].T, col2(p['bg']),
        p['Wo'].T,
        col2(p['ln1_g']), col2(p['ln1_b']), col2(p['ln2_g']), col2(p['ln2_b']),
        p['Wt1'], row2(p['bt1']), p['Wt2'], row2(p['bt2']),
        p['Wc1'].T, col2(p['bc1']), p['Wc2'].T, col2(p['bc2']),
        p['Wh1'].T, col2(p['bh1']), p['Wh2'].T, row2(p['bh2']),
    )

    def const_spec(a):
        nd = a.ndim
        return pl.BlockSpec(a.shape, lambda i, _nd=nd: (0,) * _nd)

    in_specs = [pl.BlockSpec((_BB, _N, 1), lambda i: (i, 0, 0)),
                pl.BlockSpec((_BB, 1, _N), lambda i: (i, 0, 0))]
    in_specs += [const_spec(a) for a in ins[2:]]

    out = pl.pallas_call(
        _tabnsa_kernel,
        grid=(_BATCH // _BB,),
        in_specs=in_specs,
        out_specs=pl.BlockSpec((_BB, 1, 1), lambda i: (i, 0, 0)),
        out_shape=jax.ShapeDtypeStruct((_BATCH, 1, 1), jnp.float32),
    )(*ins)
    return out.reshape(_BATCH, 1)


# speedup vs baseline: 1.8453x; 1.8453x over previous
import jax
import jax.numpy as jnp
from jax.experimental import pallas as pl

_DIM = 64
_HEADS = 8
_DH = 64
_INNER = _HEADS * _DH
_N = 128
_CBS = 4
_NC = _N // _CBS
_WIN = 2
_FF = 256
_BATCH = 512

_BB = 2
_NEG = -1e9
_SCALE = _DH ** -0.5
_HI = jax.lax.Precision.HIGHEST


def _ln_t(xt, g_col, b_col, eps=1e-5):
    m = xt.mean(0, keepdims=True)
    v = ((xt - m) ** 2).mean(0, keepdims=True)
    return (xt - m) / jnp.sqrt(v + eps) * g_col + b_col


def _tabnsa_kernel(
    x_ref, x2_ref,
    w_emb_ref, b_emb_ref, w_embt_ref, b_embt_ref,
    wqst_ref, wk_ref, wvt_ref,
    wgt_ref, bgt_ref,
    wot_ref,
    ln1g_ref, ln1b_ref, ln2g_ref, ln2b_ref,
    wt1_ref, bt1r_ref, wt2_ref, bt2r_ref,
    wc1t_ref, bc1c_ref, wc2t_ref, bc2c_ref,
    wh1t_ref, bh1c_ref, wh2t_ref, bh2_ref,
    o_ref,
):
    f32 = jnp.float32
    kk = jax.lax.broadcasted_iota(jnp.int32, (_N, _N), 0)
    qq = jax.lax.broadcasted_iota(jnp.int32, (_N, _N), 1)
    causal = kk <= qq
    winm = causal & ((qq - kk) < _WIN)
    cmask = ((kk // _CBS) * _CBS + (_CBS - 1)) <= qq
    negfill = jnp.where(qq[0:1, :] < (_CBS - 1), 0.0, _NEG)
    repm = jnp.where((kk // _CBS) == (qq // _CBS), 0.25, 0.0).astype(f32)
    neg_big = jnp.finfo(f32).min

    xcol = x_ref[...].reshape(_BB * _N, 1)
    e = xcol * w_emb_ref[...] + b_emb_ref[...]
    et = jnp.concatenate(
        [w_embt_ref[...] * x2_ref[b] + b_embt_ref[...] for b in range(_BB)],
        axis=1)

    k_all = jnp.dot(e, wk_ref[...])
    qt_all = jnp.dot(wqst_ref[...], et)
    vt_all = jnp.dot(wvt_ref[...], et)
    gt_all = jax.nn.sigmoid(jnp.dot(wgt_ref[...], et) + bgt_ref[...])

    outs = []
    for b in range(_BB):
        r0 = b * _N
        kb = k_all[r0:r0 + _N]
        qtb = qt_all[:, r0:r0 + _N]
        vtb = vt_all[:, r0:r0 + _N]
        gtb = gt_all[:, r0:r0 + _N]
        kcb = jnp.dot(repm, kb, precision=_HI)
        vctb = jnp.dot(vtb, repm)

        attn_heads = []
        for h in range(_HEADS):
            s0 = h * _DH
            qt = qtb[s0:s0 + _DH]
            vt = vtb[s0:s0 + _DH]
            vct = vctb[s0:s0 + _DH]
            s_full = jnp.dot(kb[:, s0:s0 + _DH], qt)
            sc_tok = jnp.dot(kcb[:, s0:s0 + _DH], qt)

            sc_m = jnp.where(cmask, sc_tok, negfill)
            ec = jnp.exp(sc_m)
            dc = jnp.sum(ec, axis=0, keepdims=True)

            m1 = jnp.max(sc_m, axis=0, keepdims=True)
            sc_ne = jnp.where(sc_m == m1, neg_big, sc_m)
            m2 = jnp.max(sc_ne, axis=0, keepdims=True)
            fsel = (sc_m == m1) | (sc_m == m2)

            e_c = jnp.exp(jnp.where(causal, s_full, _NEG))

            w_f = jnp.where(fsel, e_c, 0.0)
            w_w = jnp.where(winm, e_c, 0.0)
            d_f = jnp.sum(w_f, axis=0, keepdims=True)
            d_w = jnp.sum(w_w, axis=0, keepdims=True)

            g0 = gtb[h:h + 1]
            g1 = gtb[_HEADS + h:_HEADS + h + 1]
            g2 = gtb[2 * _HEADS + h:2 * _HEADS + h + 1]
            p_fw = (g1 / d_f) * w_f + (g2 / d_w) * w_w
            p_c = (g0 / dc) * ec
            attn_heads.append(jnp.dot(vt, p_fw) + jnp.dot(vct, p_c))
        outs.append(jnp.concatenate(attn_heads, axis=0))

    attn_t = jnp.concatenate(outs, axis=1)
    attn_out = jnp.dot(wot_ref[...], attn_t)

    t = _ln_t(et, ln1g_ref[...], ln1b_ref[...])
    tmix = []
    for b in range(_BB):
        tb = t[:, b * _N:(b + 1) * _N]
        a1 = jax.nn.gelu(jnp.dot(tb, wt1_ref[...]) + bt1r_ref[...])
        tmix.append(jnp.dot(a1, wt2_ref[...]) + bt2r_ref[...])
    y = et + jnp.concatenate(tmix, axis=1)
    c_in = _ln_t(y, ln2g_ref[...], ln2b_ref[...])
    c1 = jax.nn.gelu(jnp.dot(wc1t_ref[...], c_in) + bc1c_ref[...])
    cmix = jnp.dot(wc2t_ref[...], c1) + bc2c_ref[...]
    mix = y + cmix

    s_all = attn_out + mix
    pooled = jnp.concatenate(
        [jnp.mean(s_all[:, b * _N:(b + 1) * _N], axis=1, keepdims=True)
         for b in range(_BB)], axis=1)
    h1 = jax.nn.gelu(jnp.dot(wh1t_ref[...], pooled) + bh1c_ref[...])
    out_t = jnp.dot(wh2t_ref[...], h1) + bh2_ref[...]
    out = jnp.concatenate([out_t[:, b:b + 1] for b in range(_BB)], axis=0)
    o_ref[...] = out.reshape(_BB, 1, 1)


@jax.jit
def kernel(x, params):
    p = params
    xr = x.reshape(_BATCH, _N, 1)
    xr2 = x.reshape(_BATCH, 1, _N)
    row2 = lambda a: a.reshape(1, -1)
    col2 = lambda a: a.reshape(-1, 1)
    ins = (
        xr, xr2,
        p['W_emb'], row2(p['b_emb']), col2(p['W_emb']), col2(p['b_emb']),
        (p['Wq'] * _SCALE).T, p['Wk'], p['Wv'].T,
        p['Wg'].T, col2(p['bg']),
        p['Wo'].T,
        col2(p['ln1_g']), col2(p['ln1_b']), col2(p['ln2_g']), col2(p['ln2_b']),
        p['Wt1'], row2(p['bt1']), p['Wt2'], row2(p['bt2']),
        p['Wc1'].T, col2(p['bc1']), p['Wc2'].T, col2(p['bc2']),
        p['Wh1'].T, col2(p['bh1']), p['Wh2'].T, row2(p['bh2']),
    )

    def const_spec(a):
        nd = a.ndim
        return pl.BlockSpec(a.shape, lambda i, _nd=nd: (0,) * _nd)

    in_specs = [pl.BlockSpec((_BB, _N, 1), lambda i: (i, 0, 0)),
                pl.BlockSpec((_BB, 1, _N), lambda i: (i, 0, 0))]
    in_specs += [const_spec(a) for a in ins[2:]]

    out = pl.pallas_call(
        _tabnsa_kernel,
        grid=(_BATCH // _BB,),
        in_specs=in_specs,
        out_specs=pl.BlockSpec((_BB, 1, 1), lambda i: (i, 0, 0)),
        out_shape=jax.ShapeDtypeStruct((_BATCH, 1, 1), jnp.float32),
    )(*ins)
    return out.reshape(_BATCH, 1)


# speedup vs baseline: 2.3999x; 1.3005x over previous
import jax
import jax.numpy as jnp
from jax.experimental import pallas as pl

_DIM = 64
_HEADS = 8
_DH = 64
_INNER = _HEADS * _DH
_N = 128
_CBS = 4
_NC = _N // _CBS
_WIN = 2
_FF = 256
_BATCH = 512

_BB = 4
_NEG = -1e9
_SCALE = _DH ** -0.5
_HI = jax.lax.Precision.HIGHEST


def _ln_t(xt, g_col, b_col, eps=1e-5):
    m = xt.mean(0, keepdims=True)
    v = ((xt - m) ** 2).mean(0, keepdims=True)
    return (xt - m) / jnp.sqrt(v + eps) * g_col + b_col


def _tabnsa_kernel(
    x_ref, x2_ref,
    w_emb_ref, b_emb_ref, w_embt_ref, b_embt_ref,
    wqst_ref, wk_ref, wvt_ref,
    wgt_ref, bgt_ref,
    wot_ref,
    ln1g_ref, ln1b_ref, ln2g_ref, ln2b_ref,
    wt1_ref, bt1r_ref, wt2_ref, bt2r_ref,
    wc1t_ref, bc1c_ref, wc2t_ref, bc2c_ref,
    wh1t_ref, bh1c_ref, wh2t_ref, bh2_ref,
    o_ref,
):
    f32 = jnp.float32
    kk = jax.lax.broadcasted_iota(jnp.int32, (_N, _N), 0)
    qq = jax.lax.broadcasted_iota(jnp.int32, (_N, _N), 1)
    causal = kk <= qq
    winm = causal & ((qq - kk) < _WIN)
    cmask = ((kk // _CBS) * _CBS + (_CBS - 1)) <= qq
    negfill = jnp.where(qq[0:1, :] < (_CBS - 1), 0.0, _NEG)
    repm = jnp.where((kk // _CBS) == (qq // _CBS), 0.25, 0.0).astype(f32)
    neg_big = jnp.finfo(f32).min

    xcol = x_ref[...].reshape(_BB * _N, 1)
    e = xcol * w_emb_ref[...] + b_emb_ref[...]
    et = jnp.concatenate(
        [w_embt_ref[...] * x2_ref[b] + b_embt_ref[...] for b in range(_BB)],
        axis=1)

    k_all = jnp.dot(e, wk_ref[...])
    qt_all = jnp.dot(wqst_ref[...], et)
    vt_all = jnp.dot(wvt_ref[...], et)
    gt_all = jax.nn.sigmoid(jnp.dot(wgt_ref[...], et) + bgt_ref[...])

    outs = []
    for b in range(_BB):
        r0 = b * _N
        kb = k_all[r0:r0 + _N]
        qtb = qt_all[:, r0:r0 + _N]
        vtb = vt_all[:, r0:r0 + _N]
        gtb = gt_all[:, r0:r0 + _N]
        kcb = jnp.dot(repm, kb, precision=_HI)
        vctb = jnp.dot(vtb, repm)

        attn_heads = []
        for h in range(_HEADS):
            s0 = h * _DH
            qt = qtb[s0:s0 + _DH]
            vt = vtb[s0:s0 + _DH]
            vct = vctb[s0:s0 + _DH]
            s_full = jnp.dot(kb[:, s0:s0 + _DH], qt)
            sc_tok = jnp.dot(kcb[:, s0:s0 + _DH], qt)

            sc_m = jnp.where(cmask, sc_tok, negfill)
            ec = jnp.exp(sc_m)
            dc = jnp.sum(ec, axis=0, keepdims=True)

            m1 = jnp.max(sc_m, axis=0, keepdims=True)
            sc_ne = jnp.where(sc_m == m1, neg_big, sc_m)
            m2 = jnp.max(sc_ne, axis=0, keepdims=True)
            fsel = (sc_m == m1) | (sc_m == m2)

            e_c = jnp.exp(jnp.where(causal, s_full, _NEG))

            w_f = jnp.where(fsel, e_c, 0.0)
            w_w = jnp.where(winm, e_c, 0.0)
            d_f = jnp.sum(w_f, axis=0, keepdims=True)
            d_w = jnp.sum(w_w, axis=0, keepdims=True)

            g0 = gtb[h:h + 1]
            g1 = gtb[_HEADS + h:_HEADS + h + 1]
            g2 = gtb[2 * _HEADS + h:2 * _HEADS + h + 1]
            p_fw = (g1 / d_f) * w_f + (g2 / d_w) * w_w
            p_c = (g0 / dc) * ec
            attn_heads.append(jnp.dot(vt, p_fw) + jnp.dot(vct, p_c))
        outs.append(jnp.concatenate(attn_heads, axis=0))

    attn_t = jnp.concatenate(outs, axis=1)
    attn_out = jnp.dot(wot_ref[...], attn_t)

    t = _ln_t(et, ln1g_ref[...], ln1b_ref[...])
    tmix = []
    for b in range(_BB):
        tb = t[:, b * _N:(b + 1) * _N]
        a1 = jax.nn.gelu(jnp.dot(tb, wt1_ref[...]) + bt1r_ref[...])
        tmix.append(jnp.dot(a1, wt2_ref[...]) + bt2r_ref[...])
    y = et + jnp.concatenate(tmix, axis=1)
    c_in = _ln_t(y, ln2g_ref[...], ln2b_ref[...])
    c1 = jax.nn.gelu(jnp.dot(wc1t_ref[...], c_in) + bc1c_ref[...])
    cmix = jnp.dot(wc2t_ref[...], c1) + bc2c_ref[...]
    mix = y + cmix

    s_all = attn_out + mix
    pooled = jnp.concatenate(
        [jnp.mean(s_all[:, b * _N:(b + 1) * _N], axis=1, keepdims=True)
         for b in range(_BB)], axis=1)
    h1 = jax.nn.gelu(jnp.dot(wh1t_ref[...], pooled) + bh1c_ref[...])
    out_t = jnp.dot(wh2t_ref[...], h1) + bh2_ref[...]
    out = jnp.concatenate([out_t[:, b:b + 1] for b in range(_BB)], axis=0)
    o_ref[...] = out.reshape(_BB, 1, 1)


@jax.jit
def kernel(x, params):
    p = params
    xr = x.reshape(_BATCH, _N, 1)
    xr2 = x.reshape(_BATCH, 1, _N)
    row2 = lambda a: a.reshape(1, -1)
    col2 = lambda a: a.reshape(-1, 1)
    ins = (
        xr, xr2,
        p['W_emb'], row2(p['b_emb']), col2(p['W_emb']), col2(p['b_emb']),
        (p['Wq'] * _SCALE).T, p['Wk'], p['Wv'].T,
        p['Wg'].T, col2(p['bg']),
        p['Wo'].T,
        col2(p['ln1_g']), col2(p['ln1_b']), col2(p['ln2_g']), col2(p['ln2_b']),
        p['Wt1'], row2(p['bt1']), p['Wt2'], row2(p['bt2']),
        p['Wc1'].T, col2(p['bc1']), p['Wc2'].T, col2(p['bc2']),
        p['Wh1'].T, col2(p['bh1']), p['Wh2'].T, row2(p['bh2']),
    )

    def const_spec(a):
        nd = a.ndim
        return pl.BlockSpec(a.shape, lambda i, _nd=nd: (0,) * _nd)

    in_specs = [pl.BlockSpec((_BB, _N, 1), lambda i: (i, 0, 0)),
                pl.BlockSpec((_BB, 1, _N), lambda i: (i, 0, 0))]
    in_specs += [const_spec(a) for a in ins[2:]]

    out = pl.pallas_call(
        _tabnsa_kernel,
        grid=(_BATCH // _BB,),
        in_specs=in_specs,
        out_specs=pl.BlockSpec((_BB, 1, 1), lambda i: (i, 0, 0)),
        out_shape=jax.ShapeDtypeStruct((_BATCH, 1, 1), jnp.float32),
    )(*ins)
    return out.reshape(_BATCH, 1)


# speedup vs baseline: 2.7273x; 1.1364x over previous
import jax
import jax.numpy as jnp
from jax.experimental import pallas as pl

_DIM = 64
_HEADS = 8
_DH = 64
_INNER = _HEADS * _DH
_N = 128
_CBS = 4
_NC = _N // _CBS
_WIN = 2
_FF = 256
_BATCH = 512

_BB = 8
_NEG = -1e9
_SCALE = _DH ** -0.5
_HI = jax.lax.Precision.HIGHEST


def _ln_t(xt, g_col, b_col, eps=1e-5):
    m = xt.mean(0, keepdims=True)
    v = ((xt - m) ** 2).mean(0, keepdims=True)
    return (xt - m) / jnp.sqrt(v + eps) * g_col + b_col


def _tabnsa_kernel(
    x_ref, x2_ref,
    w_emb_ref, b_emb_ref, w_embt_ref, b_embt_ref,
    wqst_ref, wk_ref, wvt_ref,
    wgt_ref, bgt_ref,
    wot_ref,
    ln1g_ref, ln1b_ref, ln2g_ref, ln2b_ref,
    wt1_ref, bt1r_ref, wt2_ref, bt2r_ref,
    wc1t_ref, bc1c_ref, wc2t_ref, bc2c_ref,
    wh1t_ref, bh1c_ref, wh2t_ref, bh2_ref,
    o_ref,
):
    f32 = jnp.float32
    kk = jax.lax.broadcasted_iota(jnp.int32, (_N, _N), 0)
    qq = jax.lax.broadcasted_iota(jnp.int32, (_N, _N), 1)
    causal = kk <= qq
    winm = causal & ((qq - kk) < _WIN)
    cmask = ((kk // _CBS) * _CBS + (_CBS - 1)) <= qq
    negfill = jnp.where(qq[0:1, :] < (_CBS - 1), 0.0, _NEG)
    repm = jnp.where((kk // _CBS) == (qq // _CBS), 0.25, 0.0).astype(f32)
    neg_big = jnp.finfo(f32).min

    xcol = x_ref[...].reshape(_BB * _N, 1)
    e = xcol * w_emb_ref[...] + b_emb_ref[...]
    et = jnp.concatenate(
        [w_embt_ref[...] * x2_ref[b] + b_embt_ref[...] for b in range(_BB)],
        axis=1)

    k_all = jnp.dot(e, wk_ref[...])
    qt_all = jnp.dot(wqst_ref[...], et)
    vt_all = jnp.dot(wvt_ref[...], et)
    gt_all = jax.nn.sigmoid(jnp.dot(wgt_ref[...], et) + bgt_ref[...])

    outs = []
    for b in range(_BB):
        r0 = b * _N
        kb = k_all[r0:r0 + _N]
        qtb = qt_all[:, r0:r0 + _N]
        vtb = vt_all[:, r0:r0 + _N]
        gtb = gt_all[:, r0:r0 + _N]
        kcb = jnp.dot(repm, kb, precision=_HI)
        vctb = jnp.dot(vtb, repm)

        attn_heads = []
        for h in range(_HEADS):
            s0 = h * _DH
            qt = qtb[s0:s0 + _DH]
            vt = vtb[s0:s0 + _DH]
            vct = vctb[s0:s0 + _DH]
            s_full = jnp.dot(kb[:, s0:s0 + _DH], qt)
            sc_tok = jnp.dot(kcb[:, s0:s0 + _DH], qt)

            sc_m = jnp.where(cmask, sc_tok, negfill)
            ec = jnp.exp(sc_m)
            dc = jnp.sum(ec, axis=0, keepdims=True)

            m1 = jnp.max(sc_m, axis=0, keepdims=True)
            sc_ne = jnp.where(sc_m == m1, neg_big, sc_m)
            m2 = jnp.max(sc_ne, axis=0, keepdims=True)
            fsel = (sc_m == m1) | (sc_m == m2)

            e_c = jnp.exp(jnp.where(causal, s_full, _NEG))

            w_f = jnp.where(fsel, e_c, 0.0)
            w_w = jnp.where(winm, e_c, 0.0)
            d_f = jnp.sum(w_f, axis=0, keepdims=True)
            d_w = jnp.sum(w_w, axis=0, keepdims=True)

            g0 = gtb[h:h + 1]
            g1 = gtb[_HEADS + h:_HEADS + h + 1]
            g2 = gtb[2 * _HEADS + h:2 * _HEADS + h + 1]
            p_fw = (g1 / d_f) * w_f + (g2 / d_w) * w_w
            p_c = (g0 / dc) * ec
            attn_heads.append(jnp.dot(vt, p_fw) + jnp.dot(vct, p_c))
        outs.append(jnp.concatenate(attn_heads, axis=0))

    attn_t = jnp.concatenate(outs, axis=1)
    attn_out = jnp.dot(wot_ref[...], attn_t)

    t = _ln_t(et, ln1g_ref[...], ln1b_ref[...])
    tmix = []
    for b in range(_BB):
        tb = t[:, b * _N:(b + 1) * _N]
        a1 = jax.nn.gelu(jnp.dot(tb, wt1_ref[...]) + bt1r_ref[...])
        tmix.append(jnp.dot(a1, wt2_ref[...]) + bt2r_ref[...])
    y = et + jnp.concatenate(tmix, axis=1)
    c_in = _ln_t(y, ln2g_ref[...], ln2b_ref[...])
    c1 = jax.nn.gelu(jnp.dot(wc1t_ref[...], c_in) + bc1c_ref[...])
    cmix = jnp.dot(wc2t_ref[...], c1) + bc2c_ref[...]
    mix = y + cmix

    s_all = attn_out + mix
    pooled = jnp.concatenate(
        [jnp.mean(s_all[:, b * _N:(b + 1) * _N], axis=1, keepdims=True)
         for b in range(_BB)], axis=1)
    h1 = jax.nn.gelu(jnp.dot(wh1t_ref[...], pooled) + bh1c_ref[...])
    out_t = jnp.dot(wh2t_ref[...], h1) + bh2_ref[...]
    out = jnp.concatenate([out_t[:, b:b + 1] for b in range(_BB)], axis=0)
    o_ref[...] = out.reshape(_BB, 1, 1)


@jax.jit
def kernel(x, params):
    p = params
    xr = x.reshape(_BATCH, _N, 1)
    xr2 = x.reshape(_BATCH, 1, _N)
    row2 = lambda a: a.reshape(1, -1)
    col2 = lambda a: a.reshape(-1, 1)
    ins = (
        xr, xr2,
        p['W_emb'], row2(p['b_emb']), col2(p['W_emb']), col2(p['b_emb']),
        (p['Wq'] * _SCALE).T, p['Wk'], p['Wv'].T,
        p['Wg'].T, col2(p['bg']),
        p['Wo'].T,
        col2(p['ln1_g']), col2(p['ln1_b']), col2(p['ln2_g']), col2(p['ln2_b']),
        p['Wt1'], row2(p['bt1']), p['Wt2'], row2(p['bt2']),
        p['Wc1'].T, col2(p['bc1']), p['Wc2'].T, col2(p['bc2']),
        p['Wh1'].T, col2(p['bh1']), p['Wh2'].T, row2(p['bh2']),
    )

    def const_spec(a):
        nd = a.ndim
        return pl.BlockSpec(a.shape, lambda i, _nd=nd: (0,) * _nd)

    in_specs = [pl.BlockSpec((_BB, _N, 1), lambda i: (i, 0, 0)),
                pl.BlockSpec((_BB, 1, _N), lambda i: (i, 0, 0))]
    in_specs += [const_spec(a) for a in ins[2:]]

    out = pl.pallas_call(
        _tabnsa_kernel,
        grid=(_BATCH // _BB,),
        in_specs=in_specs,
        out_specs=pl.BlockSpec((_BB, 1, 1), lambda i: (i, 0, 0)),
        out_shape=jax.ShapeDtypeStruct((_BATCH, 1, 1), jnp.float32),
    )(*ins)
    return out.reshape(_BATCH, 1)
